# Initial kernel scaffold; baseline (speedup 1.0000x reference)
import jax
import jax.numpy as jnp
from jax.experimental import pallas as pl


def kernel(hidden_states, W):
    raise NotImplementedError("write your pallas kernel here")


# speedup vs baseline: 1.1404x; 1.1404x over previous
import functools

import jax
import jax.numpy as jnp
from jax.experimental import pallas as pl

_CAPACITY = 0.125


def _matvec_body(h_ref, w_ref, out_ref):
    out_ref[...] = jax.lax.dot_general(
        w_ref[...], h_ref[...],
        (((1,), (1,)), ((), ())),
        preferred_element_type=jnp.float32,
    )


def _select_body(k, s_ref, mask_ref):
    s = s_ref[...]
    B, S = s.shape
    b = jax.lax.bitcast_convert_type(s, jnp.int32)
    m = jax.lax.shift_right_arithmetic(b, 31)
    key = b ^ (m & jnp.int32(0x7FFFFFFF))
    MIN32 = jnp.int32(-(2 ** 31))

    def bit_step(i, cur_u):
        bit = jax.lax.shift_left(jnp.int32(1), 31 - i)
        trial_u = cur_u | bit
        trial_s = trial_u ^ MIN32
        cnt = jnp.sum((key >= trial_s).astype(jnp.int32), axis=1,
                      keepdims=True)
        return jnp.where(cnt >= k, trial_u, cur_u)

    cur_u = jax.lax.fori_loop(0, 32, bit_step,
                              jnp.zeros((B, 1), jnp.int32))
    T = cur_u ^ MIN32
    gt = key > T
    eq = key == T
    cnt_gt = jnp.sum(gt.astype(jnp.int32), axis=1, keepdims=True)
    need = k - cnt_gt
    idx = jax.lax.broadcasted_iota(jnp.int32, (B, S), 1)

    def idx_step(i, cur):
        trial = cur | jax.lax.shift_left(jnp.int32(1), 13 - i)
        cnt = jnp.sum((eq & (idx < trial)).astype(jnp.int32), axis=1,
                      keepdims=True)
        return jnp.where(cnt <= need, trial, cur)

    tbound = jax.lax.fori_loop(0, 14, idx_step,
                               jnp.zeros((B, 1), jnp.int32))
    mask_ref[...] = (gt | (eq & (idx < tbound))).astype(jnp.float32)


def kernel(hidden_states, W):
    B, S, D = hidden_states.shape
    k = int(_CAPACITY * S)
    h2 = hidden_states.reshape(B * S, D)
    BLK = 512
    grid = (B * S) // BLK
    scores_row = pl.pallas_call(
        _matvec_body,
        grid=(grid,),
        in_specs=[
            pl.BlockSpec((BLK, D), lambda i: (i, 0)),
            pl.BlockSpec((1, D), lambda i: (0, 0)),
        ],
        out_specs=pl.BlockSpec((1, BLK), lambda i: (0, i)),
        out_shape=jax.ShapeDtypeStruct((1, B * S), jnp.float32),
    )(h2, W)
    scores = scores_row.reshape(B, S)
    if k >= S:
        return (jnp.ones_like(scores), scores)
    mask = pl.pallas_call(
        functools.partial(_select_body, k),
        out_shape=jax.ShapeDtypeStruct((B, S), jnp.float32),
    )(scores)
    return (mask, scores)
